# Initial kernel scaffold; baseline (speedup 1.0000x reference)
import jax
import jax.numpy as jnp
from jax.experimental import pallas as pl


def kernel(x, xl, W, b):
    raise NotImplementedError("write your pallas kernel here")


# speedup vs baseline: 1.9145x; 1.9145x over previous
import functools

import jax
import jax.numpy as jnp
from jax.experimental import pallas as pl

_ROWS = 256


def _logsoftmax_kernel(x_ref, wt_ref, b_ref, o_ref):
    logits = (
        jnp.dot(x_ref[...], wt_ref[...], preferred_element_type=jnp.float32)
        + b_ref[...]
    )
    m = jnp.max(logits, axis=1, keepdims=True)
    lse = jnp.log(jnp.sum(jnp.exp(logits - m), axis=1, keepdims=True))
    o_ref[...] = logits - m - lse


@jax.jit
def kernel(x, xl, W, b):
    B, T, D = x.shape
    V = W.shape[0]
    rows = B * T
    x2 = x.reshape(rows, D)
    wt = W.T
    b2 = b.reshape(1, V)

    out = pl.pallas_call(
        _logsoftmax_kernel,
        grid=(rows // _ROWS,),
        in_specs=[
            pl.BlockSpec((_ROWS, D), lambda i: (i, 0)),
            pl.BlockSpec((D, V), lambda i: (0, 0)),
            pl.BlockSpec((1, V), lambda i: (0, 0)),
        ],
        out_specs=pl.BlockSpec((_ROWS, V), lambda i: (i, 0)),
        out_shape=jax.ShapeDtypeStruct((rows, V), jnp.float32),
    )(x2, wt, b2)
    return out.reshape(B, T, V)
